# baseline (speedup 1.0000x reference)
import functools

import jax
import jax.numpy as jnp
from jax import lax
from jax.experimental import pallas as pl
from jax.experimental.pallas import tpu as pltpu
from jax.experimental.pallas import tpu_sc as plsc

NC = 2
NS = 16
NW = NC * NS
L = 16
LQ = 4
BG = NW // LQ
CHUNK = 32
RB = 4
LN_EPS = 1e-12


def _rsqrt(x):
    i = lax.bitcast_convert_type(x, jnp.int32)
    i = jnp.full(i.shape, 0x5F3759DF, jnp.int32) - lax.shift_right_logical(i, 1)
    y = lax.bitcast_convert_type(i, jnp.float32)
    y = y * (1.5 - 0.5 * x * y * y)
    y = y * (1.5 - 0.5 * x * y * y)
    y = y * (1.5 - 0.5 * x * y * y)
    return y


_GATHER_DNUMS = lax.GatherDimensionNumbers(
    offset_dims=(), collapsed_slice_dims=(0,), start_index_map=(0,))


def _rotate(x, sh):
    perm = lax.bitwise_and(lax.iota(jnp.int32, L) + sh, L - 1)
    return lax.gather(x, perm[:, None], _GATHER_DNUMS, (1,),
                      mode=lax.GatherScatterMode.PROMISE_IN_BOUNDS)


def _lane_total(x):
    for sh in (8, 4, 2, 1):
        x = x + _rotate(x, sh)
    return x


def _fold_tt(pe_ref, tt_ref, o_ref):
    o_ref[...] = pe_ref[...] + tt_ref[0][None, None, :]


def _make_sc_kernel(n_seq, seq_len, d):
    ng = d // L
    pos_per_w = seq_len // LQ
    rows_per_w = (n_seq // BG) * pos_per_w
    n_chunks = rows_per_w // CHUNK

    def body(idx_hbm, oidx_hbm, word_hbm, pe_hbm, gamma_hbm, beta_hbm,
             out_hbm, idx_v, oidx_v, rows0_v, rows1_v, pe_v,
             gamma_v, beta_v, gsem0, gsem1, ssem0, ssem1):
        bufs = (rows0_v, rows1_v)
        gsems = (gsem0, gsem1)
        ssems = (ssem0, ssem1)
        wid = lax.axis_index("s") * NC + lax.axis_index("c")
        lg = lax.rem(wid, LQ)

        pltpu.sync_copy(idx_hbm.at[wid], idx_v)
        pltpu.sync_copy(oidx_hbm.at[wid], oidx_v)
        pltpu.sync_copy(pe_hbm.at[lg], pe_v)
        pltpu.sync_copy(gamma_hbm, gamma_v)
        pltpu.sync_copy(beta_hbm, beta_v)

        def compute_chunk(buf, j):
            @plsc.parallel_loop(0, CHUNK, RB)
            def _(r0):
                rows = [r0 + ri for ri in range(RB)]
                poss = [lax.rem(j * CHUNK + r, pos_per_w) for r in rows]
                accs = [None] * RB
                acc2s = [None] * RB
                for g in range(ng):
                    sl = pl.ds(g * L, L)
                    for ri in range(RB):
                        h = buf[rows[ri], sl] + pe_v[poss[ri], sl]
                        buf[rows[ri], sl] = h
                        accs[ri] = h if accs[ri] is None else accs[ri] + h
                        acc2s[ri] = (h * h if acc2s[ri] is None
                                     else acc2s[ri] + h * h)
                mus = [_lane_total(a) * (1.0 / d) for a in accs]
                rss = [_rsqrt(_lane_total(a2) * (1.0 / d) - m * m + LN_EPS)
                       for a2, m in zip(acc2s, mus)]
                for g in range(ng):
                    sl = pl.ds(g * L, L)
                    ga = gamma_v[sl]
                    be = beta_v[sl]
                    for ri in range(RB):
                        o = (buf[rows[ri], sl] - mus[ri]) * rss[ri]
                        buf[rows[ri], sl] = o * ga + be

        def start_gather(c, b):
            pltpu.async_copy(word_hbm.at[idx_v.at[c]], bufs[b], gsems[b])

        def wait_gather(c, b):
            pltpu.make_async_copy(word_hbm.at[idx_v.at[c]], bufs[b],
                                  gsems[b]).wait()

        def start_store(c, b):
            pltpu.async_copy(bufs[b], out_hbm.at[oidx_v.at[c]], ssems[b])

        def wait_store(c, b):
            pltpu.make_async_copy(bufs[b], out_hbm.at[oidx_v.at[c]],
                                  ssems[b]).wait()

        start_gather(0, 0)

        def outer(i, _):
            c0 = i * 2
            for b in range(2):
                c = c0 + b

                @pl.when(c + 1 < n_chunks)
                def _():
                    @pl.when(c >= 1)
                    def _():
                        wait_store(c - 1, 1 - b)
                    start_gather(c + 1, 1 - b)

                wait_gather(c, b)
                compute_chunk(bufs[b], c)
                start_store(c, b)
            return 0

        lax.fori_loop(0, n_chunks // 2, outer, 0)
        wait_store(n_chunks - 2, 0)
        wait_store(n_chunks - 1, 1)

    return pl.kernel(
        body,
        out_type=jax.ShapeDtypeStruct((n_seq * seq_len, d), jnp.float32),
        mesh=plsc.VectorSubcoreMesh(core_axis_name="c", subcore_axis_name="s"),
        scratch_types=[
            pltpu.VMEM((n_chunks, CHUNK), jnp.int32),
            pltpu.VMEM((n_chunks, CHUNK), jnp.int32),
            pltpu.VMEM((CHUNK, d), jnp.float32),
            pltpu.VMEM((CHUNK, d), jnp.float32),
            pltpu.VMEM((pos_per_w, d), jnp.float32),
            pltpu.VMEM((d,), jnp.float32),
            pltpu.VMEM((d,), jnp.float32),
            pltpu.SemaphoreType.DMA,
            pltpu.SemaphoreType.DMA,
            pltpu.SemaphoreType.DMA,
            pltpu.SemaphoreType.DMA,
        ],
    )


@jax.jit
def _run(x_i32, word_emb, pos_emb, tt_emb, ln_gamma, ln_beta):
    n_seq, seq_len = x_i32.shape
    d = word_emb.shape[1]
    pos_per_w = seq_len // LQ
    seq_per_w = n_seq // BG
    rows_per_w = seq_per_w * pos_per_w

    idx_t = jnp.transpose(
        x_i32.reshape(BG, seq_per_w, LQ, pos_per_w), (0, 2, 1, 3)
    ).reshape(NW, rows_per_w // CHUNK, CHUNK)

    bgs = jnp.arange(BG, dtype=jnp.int32)[:, None, None, None]
    lgs = jnp.arange(LQ, dtype=jnp.int32)[None, :, None, None]
    bis = jnp.arange(seq_per_w, dtype=jnp.int32)[None, None, :, None]
    ps = jnp.arange(pos_per_w, dtype=jnp.int32)[None, None, None, :]
    oidx = ((bgs * seq_per_w + bis) * seq_len + lgs * pos_per_w + ps)
    oidx = jnp.broadcast_to(oidx, (BG, LQ, seq_per_w, pos_per_w)).reshape(
        NW, rows_per_w // CHUNK, CHUNK)

    pe_tt = pl.pallas_call(
        _fold_tt,
        out_shape=jax.ShapeDtypeStruct((LQ, pos_per_w, d), jnp.float32),
    )(pos_emb.reshape(LQ, pos_per_w, d), tt_emb)
    sc = _make_sc_kernel(n_seq, seq_len, d)
    out = sc(idx_t, oidx, word_emb, pe_tt, ln_gamma, ln_beta)
    return out.reshape(n_seq, seq_len, d)


def kernel(x, word_emb, pos_emb, tt_emb, ln_gamma, ln_beta):
    return _run(x.astype(jnp.int32), word_emb, pos_emb, tt_emb,
                ln_gamma, ln_beta)


# speedup vs baseline: 4.4541x; 4.4541x over previous
import functools

import jax
import jax.numpy as jnp
from jax import lax
from jax.experimental import pallas as pl
from jax.experimental.pallas import tpu as pltpu
from jax.experimental.pallas import tpu_sc as plsc

NC = 2
NS = 16
NW = NC * NS
CHUNK = 32
BR = 400
LN_EPS = 1e-12


def _make_sc_gather(n_rows, d):
    rows_per_w = n_rows // NW
    n_chunks = rows_per_w // CHUNK

    def body(idx_hbm, word_hbm, out_hbm, idx_v, rows0_v, rows1_v,
             gsem0, gsem1, ssem0, ssem1):
        bufs = (rows0_v, rows1_v)
        gsems = (gsem0, gsem1)
        ssems = (ssem0, ssem1)
        wid = lax.axis_index("s") * NC + lax.axis_index("c")
        base_row = wid * rows_per_w

        pltpu.sync_copy(idx_hbm.at[wid], idx_v)

        def start_gather(c, b):
            pltpu.async_copy(word_hbm.at[idx_v.at[c]], bufs[b], gsems[b])

        def wait_gather(c, b):
            pltpu.make_async_copy(word_hbm.at[idx_v.at[c]], bufs[b],
                                  gsems[b]).wait()

        def start_store(c, b):
            pltpu.async_copy(
                bufs[b], out_hbm.at[pl.ds(base_row + c * CHUNK, CHUNK)],
                ssems[b])

        def wait_store(c, b):
            pltpu.make_async_copy(
                bufs[b], out_hbm.at[pl.ds(base_row + c * CHUNK, CHUNK)],
                ssems[b]).wait()

        start_gather(0, 0)

        def outer(i, _):
            c0 = i * 2
            for b in range(2):
                c = c0 + b

                @pl.when(c + 1 < n_chunks)
                def _():
                    @pl.when(c >= 1)
                    def _():
                        wait_store(c - 1, 1 - b)
                    start_gather(c + 1, 1 - b)

                wait_gather(c, b)
                start_store(c, b)
            return 0

        lax.fori_loop(0, n_chunks // 2, outer, 0)
        wait_store(n_chunks - 2, 0)
        wait_store(n_chunks - 1, 1)

    return pl.kernel(
        body,
        out_type=jax.ShapeDtypeStruct((n_rows, d), jnp.float32),
        mesh=plsc.VectorSubcoreMesh(core_axis_name="c", subcore_axis_name="s"),
        scratch_types=[
            pltpu.VMEM((n_chunks, CHUNK), jnp.int32),
            pltpu.VMEM((CHUNK, d), jnp.float32),
            pltpu.VMEM((CHUNK, d), jnp.float32),
            pltpu.SemaphoreType.DMA,
            pltpu.SemaphoreType.DMA,
            pltpu.SemaphoreType.DMA,
            pltpu.SemaphoreType.DMA,
        ],
    )


def _tc_ln(x_ref, pe_ref, tt_ref, g_ref, b_ref, o_ref):
    h = x_ref[...] + pe_ref[...] + tt_ref[0:1, :]
    mu = jnp.mean(h, axis=1, keepdims=True)
    var = jnp.mean((h - mu) * (h - mu), axis=1, keepdims=True)
    o_ref[...] = ((h - mu) * lax.rsqrt(var + LN_EPS)) * g_ref[0:1, :] \
        + b_ref[0:1, :]


@jax.jit
def _run(x_i32, word_emb, pos_emb, tt_emb, ln_gamma, ln_beta):
    n_seq, seq_len = x_i32.shape
    d = word_emb.shape[1]
    n_rows = n_seq * seq_len

    idx_t = x_i32.reshape(NW, n_rows // NW // CHUNK, CHUNK)
    gathered = _make_sc_gather(n_rows, d)(idx_t, word_emb)

    pe_t = jnp.tile(pos_emb, (BR // seq_len, 1))
    out = pl.pallas_call(
        _tc_ln,
        grid=(n_rows // BR,),
        in_specs=[
            pl.BlockSpec((BR, d), lambda i: (i, 0)),
            pl.BlockSpec((BR, d), lambda i: (0, 0)),
            pl.BlockSpec((2, d), lambda i: (0, 0)),
            pl.BlockSpec((1, d), lambda i: (0, 0)),
            pl.BlockSpec((1, d), lambda i: (0, 0)),
        ],
        out_specs=pl.BlockSpec((BR, d), lambda i: (i, 0)),
        out_shape=jax.ShapeDtypeStruct((n_rows, d), jnp.float32),
    )(gathered, pe_t, tt_emb, ln_gamma.reshape(1, d), ln_beta.reshape(1, d))
    return out.reshape(n_seq, seq_len, d)


def kernel(x, word_emb, pos_emb, tt_emb, ln_gamma, ln_beta):
    return _run(x.astype(jnp.int32), word_emb, pos_emb, tt_emb,
                ln_gamma, ln_beta)


# speedup vs baseline: 10.1884x; 2.2874x over previous
import functools

import jax
import jax.numpy as jnp
from jax import lax
from jax.experimental import pallas as pl
from jax.experimental.pallas import tpu as pltpu
from jax.experimental.pallas import tpu_sc as plsc

NC = 2
NS = 16
NW = NC * NS
CHUNK = 32
BR = 400
LN_EPS = 1e-12


def _make_sc_gather(n_rows, d):
    rows_per_w = n_rows // NW
    n_chunks = rows_per_w // CHUNK

    def body(idx_hbm, word_hbm, out_hbm, idx_v, rows0_v, rows1_v,
             gsem0, gsem1, ssem0, ssem1):
        bufs = (rows0_v, rows1_v)
        gsems = (gsem0, gsem1)
        ssems = (ssem0, ssem1)
        wid = lax.axis_index("s") * NC + lax.axis_index("c")
        base_row = wid * rows_per_w

        pltpu.sync_copy(idx_hbm.at[wid], idx_v)

        def start_gather(c, b):
            pltpu.async_copy(word_hbm.at[idx_v.at[c]], bufs[b], gsems[b])

        def wait_gather(c, b):
            pltpu.make_async_copy(word_hbm.at[idx_v.at[c]], bufs[b],
                                  gsems[b]).wait()

        def start_store(c, b):
            pltpu.async_copy(
                bufs[b], out_hbm.at[pl.ds(base_row + c * CHUNK, CHUNK)],
                ssems[b])

        def wait_store(c, b):
            pltpu.make_async_copy(
                bufs[b], out_hbm.at[pl.ds(base_row + c * CHUNK, CHUNK)],
                ssems[b]).wait()

        start_gather(0, 0)

        def outer(i, _):
            c0 = i * 2
            for b in range(2):
                c = c0 + b

                @pl.when(c + 1 < n_chunks)
                def _():
                    @pl.when(c >= 1)
                    def _():
                        wait_store(c - 1, 1 - b)
                    start_gather(c + 1, 1 - b)

                wait_gather(c, b)
                start_store(c, b)
            return 0

        lax.fori_loop(0, n_chunks // 2, outer, 0)
        wait_store(n_chunks - 2, 0)
        wait_store(n_chunks - 1, 1)

    return pl.kernel(
        body,
        out_type=jax.ShapeDtypeStruct((n_rows, d), jnp.float32),
        mesh=plsc.VectorSubcoreMesh(core_axis_name="c", subcore_axis_name="s"),
        scratch_types=[
            pltpu.VMEM((n_chunks, CHUNK), jnp.int32),
            pltpu.VMEM((CHUNK, d), jnp.float32),
            pltpu.VMEM((CHUNK, d), jnp.float32),
            pltpu.SemaphoreType.DMA,
            pltpu.SemaphoreType.DMA,
            pltpu.SemaphoreType.DMA,
            pltpu.SemaphoreType.DMA,
        ],
    )


def _tc_ln(x_ref, pe_ref, tt_ref, g_ref, b_ref, o_ref):
    h = x_ref[...] + pe_ref[0] + tt_ref[0:1, :]
    mu = jnp.mean(h, axis=1, keepdims=True)
    var = jnp.mean((h - mu) * (h - mu), axis=1, keepdims=True)
    o_ref[...] = ((h - mu) * lax.rsqrt(var + LN_EPS)) * g_ref[0:1, :] \
        + b_ref[0:1, :]


@jax.jit
def _run(x_i32, word_emb, pos_emb, tt_emb, ln_gamma, ln_beta):
    n_seq, seq_len = x_i32.shape
    d = word_emb.shape[1]
    n_rows = n_seq * seq_len

    idx_t = x_i32.T.reshape(NW, n_rows // NW // CHUNK, CHUNK)
    gathered = _make_sc_gather(n_rows, d)(idx_t, word_emb)

    out = pl.pallas_call(
        _tc_ln,
        grid=(seq_len,),
        in_specs=[
            pl.BlockSpec((n_seq, d), lambda i: (i, 0)),
            pl.BlockSpec((1, 1, d), lambda i: (i, 0, 0)),
            pl.BlockSpec((2, d), lambda i: (0, 0)),
            pl.BlockSpec((1, d), lambda i: (0, 0)),
            pl.BlockSpec((1, d), lambda i: (0, 0)),
        ],
        out_specs=pl.BlockSpec((n_seq, d), lambda i: (i, 0)),
        out_shape=jax.ShapeDtypeStruct((n_rows, d), jnp.float32),
    )(gathered, pos_emb.reshape(seq_len, 1, d), tt_emb,
      ln_gamma.reshape(1, d), ln_beta.reshape(1, d))
    return jnp.transpose(out.reshape(seq_len, n_seq, d), (1, 0, 2))


def kernel(x, word_emb, pos_emb, tt_emb, ln_gamma, ln_beta):
    return _run(x.astype(jnp.int32), word_emb, pos_emb, tt_emb,
                ln_gamma, ln_beta)


# speedup vs baseline: 10.3009x; 1.0110x over previous
import functools

import jax
import jax.numpy as jnp
from jax import lax
from jax.experimental import pallas as pl
from jax.experimental.pallas import tpu as pltpu
from jax.experimental.pallas import tpu_sc as plsc

NC = 2
NS = 16
NW = NC * NS
CHUNK = 16
BR = 400
LN_EPS = 1e-12


def _make_sc_gather(n_rows, d):
    rows_per_w = n_rows // NW
    n_chunks = rows_per_w // CHUNK

    def body(idx_hbm, word_hbm, out_hbm, idx_v, rows0_v, rows1_v,
             gsem0, gsem1, ssem0, ssem1):
        bufs = (rows0_v, rows1_v)
        gsems = (gsem0, gsem1)
        ssems = (ssem0, ssem1)
        wid = lax.axis_index("s") * NC + lax.axis_index("c")
        base_row = wid * rows_per_w

        pltpu.sync_copy(idx_hbm.at[wid], idx_v)

        def start_gather(c, b):
            pltpu.async_copy(word_hbm.at[idx_v.at[c]], bufs[b], gsems[b])

        def wait_gather(c, b):
            pltpu.make_async_copy(word_hbm.at[idx_v.at[c]], bufs[b],
                                  gsems[b]).wait()

        def start_store(c, b):
            pltpu.async_copy(
                bufs[b], out_hbm.at[pl.ds(base_row + c * CHUNK, CHUNK)],
                ssems[b])

        def wait_store(c, b):
            pltpu.make_async_copy(
                bufs[b], out_hbm.at[pl.ds(base_row + c * CHUNK, CHUNK)],
                ssems[b]).wait()

        start_gather(0, 0)

        def outer(i, _):
            c0 = i * 2
            for b in range(2):
                c = c0 + b

                @pl.when(c + 1 < n_chunks)
                def _():
                    @pl.when(c >= 1)
                    def _():
                        wait_store(c - 1, 1 - b)
                    start_gather(c + 1, 1 - b)

                wait_gather(c, b)
                start_store(c, b)
            return 0

        lax.fori_loop(0, n_chunks // 2, outer, 0)
        wait_store(n_chunks - 2, 0)
        wait_store(n_chunks - 1, 1)

    return pl.kernel(
        body,
        out_type=jax.ShapeDtypeStruct((n_rows, d), jnp.float32),
        mesh=plsc.VectorSubcoreMesh(core_axis_name="c", subcore_axis_name="s"),
        scratch_types=[
            pltpu.VMEM((n_chunks, CHUNK), jnp.int32),
            pltpu.VMEM((CHUNK, d), jnp.float32),
            pltpu.VMEM((CHUNK, d), jnp.float32),
            pltpu.SemaphoreType.DMA,
            pltpu.SemaphoreType.DMA,
            pltpu.SemaphoreType.DMA,
            pltpu.SemaphoreType.DMA,
        ],
    )


def _tc_ln(x_ref, pe_ref, tt_ref, g_ref, b_ref, o_ref):
    h = x_ref[...] + pe_ref[0] + tt_ref[0:1, :]
    mu = jnp.mean(h, axis=1, keepdims=True)
    var = jnp.mean((h - mu) * (h - mu), axis=1, keepdims=True)
    o_ref[...] = ((h - mu) * lax.rsqrt(var + LN_EPS)) * g_ref[0:1, :] \
        + b_ref[0:1, :]


def _tc_ln_acc(x_ref, pe_ref, tt_ref, g_ref, b_ref, prev_ref, o_ref):
    del prev_ref
    _tc_ln(x_ref, pe_ref, tt_ref, g_ref, b_ref, o_ref)


NSLICE = 4


@jax.jit
def _run(x_i32, word_emb, pos_emb, tt_emb, ln_gamma, ln_beta):
    n_seq, seq_len = x_i32.shape
    d = word_emb.shape[1]
    n_rows = n_seq * seq_len

    x_pm = x_i32.T
    pe3 = pos_emb.reshape(seq_len, 1, d)
    g2 = ln_gamma.reshape(1, d)
    b2 = ln_beta.reshape(1, d)

    sl_len = seq_len // NSLICE
    sl_rows = sl_len * n_seq
    sc = _make_sc_gather(sl_rows, d)

    gathered = [
        sc(x_pm[s * sl_len:(s + 1) * sl_len]
           .reshape(NW, sl_rows // NW // CHUNK, CHUNK), word_emb)
        for s in range(NSLICE)
    ]

    out = None
    for s in range(NSLICE):
        off = s * sl_len
        in_specs = [
            pl.BlockSpec((n_seq, d), lambda i: (i, 0)),
            pl.BlockSpec((1, 1, d), lambda i, off=off: (off + i, 0, 0)),
            pl.BlockSpec((2, d), lambda i: (0, 0)),
            pl.BlockSpec((1, d), lambda i: (0, 0)),
            pl.BlockSpec((1, d), lambda i: (0, 0)),
        ]
        out_spec = pl.BlockSpec((n_seq, d), lambda i, off=off: (off + i, 0))
        if s == 0:
            out = pl.pallas_call(
                _tc_ln,
                grid=(sl_len,),
                in_specs=in_specs,
                out_specs=out_spec,
                out_shape=jax.ShapeDtypeStruct((n_rows, d), jnp.float32),
            )(gathered[s], pe3, tt_emb, g2, b2)
        else:
            out = pl.pallas_call(
                _tc_ln_acc,
                grid=(sl_len,),
                in_specs=in_specs + [pl.BlockSpec(memory_space=pl.ANY)],
                out_specs=out_spec,
                out_shape=jax.ShapeDtypeStruct((n_rows, d), jnp.float32),
                input_output_aliases={5: 0},
            )(gathered[s], pe3, tt_emb, g2, b2, out)
    return jnp.transpose(out.reshape(seq_len, n_seq, d), (1, 0, 2))


def kernel(x, word_emb, pos_emb, tt_emb, ln_gamma, ln_beta):
    return _run(x.astype(jnp.int32), word_emb, pos_emb, tt_emb,
                ln_gamma, ln_beta)


# speedup vs baseline: 10.3775x; 1.0074x over previous
import functools

import jax
import jax.numpy as jnp
from jax import lax
from jax.experimental import pallas as pl
from jax.experimental.pallas import tpu as pltpu
from jax.experimental.pallas import tpu_sc as plsc

NC = 2
NS = 16
NW = NC * NS
CHUNK = 16
BR = 400
LN_EPS = 1e-12


def _make_sc_gather(n_rows, d):
    rows_per_w = n_rows // NW
    n_chunks = rows_per_w // CHUNK

    def body(idx_hbm, word_hbm, out_hbm, idx_v, rows0_v, rows1_v,
             gsem0, gsem1, ssem0, ssem1):
        bufs = (rows0_v, rows1_v)
        gsems = (gsem0, gsem1)
        ssems = (ssem0, ssem1)
        wid = lax.axis_index("s") * NC + lax.axis_index("c")
        base_row = wid * rows_per_w

        pltpu.sync_copy(idx_hbm.at[wid], idx_v)

        def start_gather(c, b):
            pltpu.async_copy(word_hbm.at[idx_v.at[c]], bufs[b], gsems[b])

        def wait_gather(c, b):
            pltpu.make_async_copy(word_hbm.at[idx_v.at[c]], bufs[b],
                                  gsems[b]).wait()

        def start_store(c, b):
            pltpu.async_copy(
                bufs[b], out_hbm.at[pl.ds(base_row + c * CHUNK, CHUNK)],
                ssems[b])

        def wait_store(c, b):
            pltpu.make_async_copy(
                bufs[b], out_hbm.at[pl.ds(base_row + c * CHUNK, CHUNK)],
                ssems[b]).wait()

        start_gather(0, 0)

        def outer(i, _):
            c0 = i * 2
            for b in range(2):
                c = c0 + b

                @pl.when(c + 1 < n_chunks)
                def _():
                    @pl.when(c >= 1)
                    def _():
                        wait_store(c - 1, 1 - b)
                    start_gather(c + 1, 1 - b)

                wait_gather(c, b)
                start_store(c, b)
            return 0

        lax.fori_loop(0, n_chunks // 2, outer, 0)
        wait_store(n_chunks - 2, 0)
        wait_store(n_chunks - 1, 1)

    return pl.kernel(
        body,
        out_type=jax.ShapeDtypeStruct((n_rows, d), jnp.float32),
        mesh=plsc.VectorSubcoreMesh(core_axis_name="c", subcore_axis_name="s"),
        scratch_types=[
            pltpu.VMEM((n_chunks, CHUNK), jnp.int32),
            pltpu.VMEM((CHUNK, d), jnp.float32),
            pltpu.VMEM((CHUNK, d), jnp.float32),
            pltpu.SemaphoreType.DMA,
            pltpu.SemaphoreType.DMA,
            pltpu.SemaphoreType.DMA,
            pltpu.SemaphoreType.DMA,
        ],
    )


def _tc_ln(x_ref, pe_ref, tt_ref, g_ref, b_ref, o_ref):
    h = x_ref[...] + pe_ref[0] + tt_ref[0:1, :]
    mu = jnp.mean(h, axis=1, keepdims=True)
    var = jnp.mean((h - mu) * (h - mu), axis=1, keepdims=True)
    o_ref[...] = ((h - mu) * lax.rsqrt(var + LN_EPS)) * g_ref[0:1, :] \
        + b_ref[0:1, :]


def _tc_ln_acc(x_ref, pe_ref, tt_ref, g_ref, b_ref, prev_ref, o_ref):
    del prev_ref
    _tc_ln(x_ref, pe_ref, tt_ref, g_ref, b_ref, o_ref)


NSLICE = 5


@jax.jit
def _run(x_i32, word_emb, pos_emb, tt_emb, ln_gamma, ln_beta):
    n_seq, seq_len = x_i32.shape
    d = word_emb.shape[1]
    n_rows = n_seq * seq_len

    x_pm = x_i32.T
    pe3 = pos_emb.reshape(seq_len, 1, d)
    g2 = ln_gamma.reshape(1, d)
    b2 = ln_beta.reshape(1, d)

    sl_len = seq_len // NSLICE
    sl_rows = sl_len * n_seq
    sc = _make_sc_gather(sl_rows, d)

    gathered = [
        sc(x_pm[s * sl_len:(s + 1) * sl_len]
           .reshape(NW, sl_rows // NW // CHUNK, CHUNK), word_emb)
        for s in range(NSLICE)
    ]

    out = None
    for s in range(NSLICE):
        off = s * sl_len
        in_specs = [
            pl.BlockSpec((n_seq, d), lambda i: (i, 0)),
            pl.BlockSpec((1, 1, d), lambda i, off=off: (off + i, 0, 0)),
            pl.BlockSpec((2, d), lambda i: (0, 0)),
            pl.BlockSpec((1, d), lambda i: (0, 0)),
            pl.BlockSpec((1, d), lambda i: (0, 0)),
        ]
        out_spec = pl.BlockSpec((n_seq, d), lambda i, off=off: (off + i, 0))
        if s == 0:
            out = pl.pallas_call(
                _tc_ln,
                grid=(sl_len,),
                in_specs=in_specs,
                out_specs=out_spec,
                out_shape=jax.ShapeDtypeStruct((n_rows, d), jnp.float32),
            )(gathered[s], pe3, tt_emb, g2, b2)
        else:
            out = pl.pallas_call(
                _tc_ln_acc,
                grid=(sl_len,),
                in_specs=in_specs + [pl.BlockSpec(memory_space=pl.ANY)],
                out_specs=out_spec,
                out_shape=jax.ShapeDtypeStruct((n_rows, d), jnp.float32),
                input_output_aliases={5: 0},
            )(gathered[s], pe3, tt_emb, g2, b2, out)
    return jnp.transpose(out.reshape(seq_len, n_seq, d), (1, 0, 2))


def kernel(x, word_emb, pos_emb, tt_emb, ln_gamma, ln_beta):
    return _run(x.astype(jnp.int32), word_emb, pos_emb, tt_emb,
                ln_gamma, ln_beta)
